# baseline (speedup 1.0000x reference)
import functools

import jax
import jax.numpy as jnp
import numpy as np
from jax import lax
from jax.experimental import pallas as pl
from jax.experimental.pallas import tpu as pltpu

_VMEM_LIMIT = 100 * 1024 * 1024


def _interp_mat_np(out_size, in_size):
    if out_size == 1:
        src = np.zeros((out_size,), np.float64)
    else:
        src = np.arange(out_size, dtype=np.float64) * (
            (in_size - 1) / (out_size - 1))
    i0 = np.clip(np.floor(src).astype(np.int64), 0, in_size - 1)
    i1 = np.clip(i0 + 1, 0, in_size - 1)
    frac = (src - i0).astype(np.float32)
    m = np.zeros((out_size, in_size), np.float32)
    m[np.arange(out_size), i0] += 1.0 - frac
    m[np.arange(out_size), i1] += frac
    return m


def _refine_kernel(f_ref, pm_ref, wfs_ref, wrs_ref, b_ref, kup_ref,
                   o_ref, *, H, W, NI):
    HW = H * W
    L = NI * HW
    col = lax.broadcasted_iota(jnp.int32, (1, L), 1) % W
    mask_l = col != 0
    mask_r = col != (W - 1)

    def shift_down(p):
        cout = p.shape[0]
        zw = jnp.zeros((cout, W), jnp.float32)
        pieces = []
        for i in range(NI):
            pieces += [zw, p[:, i * HW:(i + 1) * HW - W]]
        return jnp.concatenate(pieces, 1)

    def shift_up(p):
        cout = p.shape[0]
        zw = jnp.zeros((cout, W), jnp.float32)
        pieces = []
        for i in range(NI):
            pieces += [p[:, i * HW + W:(i + 1) * HW], zw]
        return jnp.concatenate(pieces, 1)

    def conv3x3(v_bf16, w2d, bias_col):
        cin = v_bf16.shape[0]
        z1 = jnp.zeros((cin, 1), jnp.bfloat16)
        xm = jnp.where(mask_l, jnp.concatenate([z1, v_bf16[:, :L - 1]], 1),
                       jnp.bfloat16(0))
        xp = jnp.where(mask_r, jnp.concatenate([v_bf16[:, 1:], z1], 1),
                       jnp.bfloat16(0))
        cols = jnp.concatenate([xm, v_bf16, xp], axis=0)
        pall = lax.dot_general(w2d, cols, (((1,), (0,)), ((), ())),
                               preferred_element_type=jnp.float32)
        cout = pall.shape[0] // 3
        out = pall[cout:2 * cout]
        out = out + shift_down(pall[0:cout])
        out = out + shift_up(pall[2 * cout:])
        return out + b_ref[:, bias_col:bias_col + 1]

    x = (jnp.concatenate([f_ref[i] for i in range(NI)], axis=1)
         if NI > 1 else f_ref[0])
    h1 = conv3x3(x, wfs_ref[...], 0)

    r = conv3x3(jnp.maximum(h1, 0.0).astype(jnp.bfloat16), wrs_ref[0], 1)
    r = conv3x3(jnp.maximum(r, 0.0).astype(jnp.bfloat16), wrs_ref[1], 2)
    s = h1 + r

    ups = [lax.dot_general(pm_ref[i], kup_ref[...], (((1,), (0,)), ((), ())),
                           preferred_element_type=jnp.float32)
           for i in range(NI)]
    m = s + (jnp.concatenate(ups, axis=1) if NI > 1 else ups[0])

    r2 = conv3x3(jnp.maximum(m, 0.0).astype(jnp.bfloat16), wrs_ref[2], 3)
    r2 = conv3x3(jnp.maximum(r2, 0.0).astype(jnp.bfloat16), wrs_ref[3], 4)
    out = m + r2
    for i in range(NI):
        o_ref[i] = out[:, i * HW:(i + 1) * HW]


def kernel(f, pm, convFS_w, convFS_b,
           resFS_conv1_w, resFS_conv1_b, resFS_conv2_w, resFS_conv2_b,
           resMM_conv1_w, resMM_conv1_b, resMM_conv2_w, resMM_conv2_b):
    N, Cin, H, W = f.shape
    _, C, h, w = pm.shape
    HW, hw = H * W, h * w
    NI = 2 if N % 2 == 0 else 1

    f_flat = f.reshape(N, Cin, HW).astype(jnp.bfloat16)
    pm_flat = pm.reshape(N, C, hw)

    wfs = (jnp.transpose(convFS_w, (0, 3, 1, 2))
           .reshape(3 * C, 3 * Cin).astype(jnp.bfloat16))
    wrs = (jnp.transpose(
        jnp.stack([resFS_conv1_w, resFS_conv2_w, resMM_conv1_w, resMM_conv2_w]),
        (0, 1, 4, 2, 3)).reshape(4, 3 * C, 3 * C).astype(jnp.bfloat16))
    bcols = jnp.stack([convFS_b, resFS_conv1_b, resFS_conv2_b,
                       resMM_conv1_b, resMM_conv2_b], axis=1)

    aht = _interp_mat_np(H, h).T
    awt = _interp_mat_np(W, w).T
    kup = jnp.asarray(
        (aht[:, None, :, None] * awt[None, :, None, :]).reshape(hw, HW))

    whole = lambda shp: pl.BlockSpec(shp, lambda i: (0,) * len(shp))

    out = pl.pallas_call(
        functools.partial(_refine_kernel, H=H, W=W, NI=NI),
        out_shape=jax.ShapeDtypeStruct((N, C, HW), jnp.float32),
        grid=(N // NI,),
        in_specs=[
            pl.BlockSpec((NI, Cin, HW), lambda i: (i, 0, 0)),
            pl.BlockSpec((NI, C, hw), lambda i: (i, 0, 0)),
            whole((3 * C, 3 * Cin)),
            whole((4, 3 * C, 3 * C)),
            whole((C, 5)),
            whole((hw, HW)),
        ],
        out_specs=pl.BlockSpec((NI, C, HW), lambda i: (i, 0, 0)),
        compiler_params=pltpu.CompilerParams(
            dimension_semantics=("arbitrary",),
            vmem_limit_bytes=_VMEM_LIMIT),
    )(f_flat, pm_flat, wfs, wrs, bcols, kup)
    return out.reshape(N, C, H, W)


# speedup vs baseline: 1.0996x; 1.0996x over previous
import functools

import jax
import jax.numpy as jnp
import numpy as np
from jax import lax
from jax.experimental import pallas as pl
from jax.experimental.pallas import tpu as pltpu

_VMEM_LIMIT = 100 * 1024 * 1024


def _interp_mat_np(out_size, in_size):
    if out_size == 1:
        src = np.zeros((out_size,), np.float64)
    else:
        src = np.arange(out_size, dtype=np.float64) * (
            (in_size - 1) / (out_size - 1))
    i0 = np.clip(np.floor(src).astype(np.int64), 0, in_size - 1)
    i1 = np.clip(i0 + 1, 0, in_size - 1)
    frac = (src - i0).astype(np.float32)
    m = np.zeros((out_size, in_size), np.float32)
    m[np.arange(out_size), i0] += 1.0 - frac
    m[np.arange(out_size), i1] += frac
    return m


def _refine_kernel(f_ref, pm_ref, wfs_ref, wrs_ref, b_ref, kup_ref,
                   o_ref, *, H, W, NI):
    HW = H * W
    L = NI * HW
    col = lax.broadcasted_iota(jnp.int32, (1, L), 1) % W
    mask_l = col != 0
    mask_r = col != (W - 1)

    def shift_down(p):
        cout = p.shape[0]
        zw = jnp.zeros((cout, W), jnp.float32)
        pieces = []
        for i in range(NI):
            pieces += [zw, p[:, i * HW:(i + 1) * HW - W]]
        return jnp.concatenate(pieces, 1)

    def shift_up(p):
        cout = p.shape[0]
        zw = jnp.zeros((cout, W), jnp.float32)
        pieces = []
        for i in range(NI):
            pieces += [p[:, i * HW + W:(i + 1) * HW], zw]
        return jnp.concatenate(pieces, 1)

    def conv3x3(v_bf16, w2d, bias_col):
        cin = v_bf16.shape[0]
        z1 = jnp.zeros((cin, 1), jnp.bfloat16)
        xm = jnp.where(mask_l, jnp.concatenate([z1, v_bf16[:, :L - 1]], 1),
                       jnp.bfloat16(0))
        xp = jnp.where(mask_r, jnp.concatenate([v_bf16[:, 1:], z1], 1),
                       jnp.bfloat16(0))
        cols = jnp.concatenate([xm, v_bf16, xp], axis=0)
        pall = lax.dot_general(w2d, cols, (((1,), (0,)), ((), ())),
                               preferred_element_type=jnp.float32)
        cout = pall.shape[0] // 3
        out = pall[cout:2 * cout]
        out = out + shift_down(pall[0:cout])
        out = out + shift_up(pall[2 * cout:])
        return out + b_ref[:, bias_col:bias_col + 1]

    x = (jnp.concatenate([f_ref[i] for i in range(NI)], axis=1)
         if NI > 1 else f_ref[0]).astype(jnp.bfloat16)
    h1 = conv3x3(x, wfs_ref[...], 0)

    r = conv3x3(jnp.maximum(h1, 0.0).astype(jnp.bfloat16), wrs_ref[0], 1)
    r = conv3x3(jnp.maximum(r, 0.0).astype(jnp.bfloat16), wrs_ref[1], 2)
    s = h1 + r

    ups = [lax.dot_general(pm_ref[i], kup_ref[...], (((1,), (0,)), ((), ())),
                           preferred_element_type=jnp.float32)
           for i in range(NI)]
    m = s + (jnp.concatenate(ups, axis=1) if NI > 1 else ups[0])

    r2 = conv3x3(jnp.maximum(m, 0.0).astype(jnp.bfloat16), wrs_ref[2], 3)
    r2 = conv3x3(jnp.maximum(r2, 0.0).astype(jnp.bfloat16), wrs_ref[3], 4)
    out = m + r2
    for i in range(NI):
        o_ref[i] = out[:, i * HW:(i + 1) * HW]


def kernel(f, pm, convFS_w, convFS_b,
           resFS_conv1_w, resFS_conv1_b, resFS_conv2_w, resFS_conv2_b,
           resMM_conv1_w, resMM_conv1_b, resMM_conv2_w, resMM_conv2_b):
    N, Cin, H, W = f.shape
    _, C, h, w = pm.shape
    HW, hw = H * W, h * w
    NI = 4 if N % 4 == 0 else (2 if N % 2 == 0 else 1)

    f_flat = f.reshape(N, Cin, HW)
    pm_flat = pm.reshape(N, C, hw)

    wfs = (jnp.transpose(convFS_w, (0, 3, 1, 2))
           .reshape(3 * C, 3 * Cin).astype(jnp.bfloat16))
    wrs = (jnp.transpose(
        jnp.stack([resFS_conv1_w, resFS_conv2_w, resMM_conv1_w, resMM_conv2_w]),
        (0, 1, 4, 2, 3)).reshape(4, 3 * C, 3 * C).astype(jnp.bfloat16))
    bcols = jnp.stack([convFS_b, resFS_conv1_b, resFS_conv2_b,
                       resMM_conv1_b, resMM_conv2_b], axis=1)

    aht = _interp_mat_np(H, h).T
    awt = _interp_mat_np(W, w).T
    kup = jnp.asarray(
        (aht[:, None, :, None] * awt[None, :, None, :]).reshape(hw, HW))

    whole = lambda shp: pl.BlockSpec(shp, lambda i: (0,) * len(shp))

    out = pl.pallas_call(
        functools.partial(_refine_kernel, H=H, W=W, NI=NI),
        out_shape=jax.ShapeDtypeStruct((N, C, HW), jnp.float32),
        grid=(N // NI,),
        in_specs=[
            pl.BlockSpec((NI, Cin, HW), lambda i: (i, 0, 0)),
            pl.BlockSpec((NI, C, hw), lambda i: (i, 0, 0)),
            whole((3 * C, 3 * Cin)),
            whole((4, 3 * C, 3 * C)),
            whole((C, 5)),
            whole((hw, HW)),
        ],
        out_specs=pl.BlockSpec((NI, C, HW), lambda i: (i, 0, 0)),
        compiler_params=pltpu.CompilerParams(
            dimension_semantics=("arbitrary",),
            vmem_limit_bytes=_VMEM_LIMIT),
    )(f_flat, pm_flat, wfs, wrs, bcols, kup)
    return out.reshape(N, C, H, W)


# speedup vs baseline: 1.1440x; 1.0404x over previous
import functools

import jax
import jax.numpy as jnp
import numpy as np
from jax import lax
from jax.experimental import pallas as pl
from jax.experimental.pallas import tpu as pltpu

_VMEM_LIMIT = 100 * 1024 * 1024


def _interp_mat_np(out_size, in_size):
    if out_size == 1:
        src = np.zeros((out_size,), np.float64)
    else:
        src = np.arange(out_size, dtype=np.float64) * (
            (in_size - 1) / (out_size - 1))
    i0 = np.clip(np.floor(src).astype(np.int64), 0, in_size - 1)
    i1 = np.clip(i0 + 1, 0, in_size - 1)
    frac = (src - i0).astype(np.float32)
    m = np.zeros((out_size, in_size), np.float32)
    m[np.arange(out_size), i0] += 1.0 - frac
    m[np.arange(out_size), i1] += frac
    return m


def _refine_kernel(f_ref, pm_ref, wfs_ref, wrs_ref, b_ref, kup_ref,
                   o_ref, *, H, W, NI):
    HW = H * W
    L = NI * HW
    col = lax.broadcasted_iota(jnp.int32, (1, L), 1) % W
    mask_l = col != 0
    mask_r = col != (W - 1)

    def shift_down(p):
        cout = p.shape[0]
        zw = jnp.zeros((cout, W), jnp.float32)
        pieces = []
        for i in range(NI):
            pieces += [zw, p[:, i * HW:(i + 1) * HW - W]]
        return jnp.concatenate(pieces, 1)

    def shift_up(p):
        cout = p.shape[0]
        zw = jnp.zeros((cout, W), jnp.float32)
        pieces = []
        for i in range(NI):
            pieces += [p[:, i * HW + W:(i + 1) * HW], zw]
        return jnp.concatenate(pieces, 1)

    def conv3x3(v_bf16, w2d, bias_col):
        cin = v_bf16.shape[0]
        z1 = jnp.zeros((cin, 1), jnp.bfloat16)
        xm = jnp.where(mask_l, jnp.concatenate([z1, v_bf16[:, :L - 1]], 1),
                       jnp.bfloat16(0))
        xp = jnp.where(mask_r, jnp.concatenate([v_bf16[:, 1:], z1], 1),
                       jnp.bfloat16(0))
        cols = jnp.concatenate([xm, v_bf16, xp], axis=0)
        pall = lax.dot_general(w2d, cols, (((1,), (0,)), ((), ())),
                               preferred_element_type=jnp.float32)
        cout = pall.shape[0] // 3
        out = pall[cout:2 * cout]
        out = out + shift_down(pall[0:cout])
        out = out + shift_up(pall[2 * cout:])
        return out + b_ref[:, bias_col:bias_col + 1]

    x = (jnp.concatenate([f_ref[i] for i in range(NI)], axis=1)
         if NI > 1 else f_ref[0]).astype(jnp.bfloat16)
    h1 = conv3x3(x, wfs_ref[...], 0)

    r = conv3x3(jnp.maximum(h1, 0.0).astype(jnp.bfloat16), wrs_ref[0], 1)
    r = conv3x3(jnp.maximum(r, 0.0).astype(jnp.bfloat16), wrs_ref[1], 2)
    s = h1 + r

    ups = [lax.dot_general(pm_ref[i], kup_ref[...], (((1,), (0,)), ((), ())),
                           preferred_element_type=jnp.float32)
           for i in range(NI)]
    m = s + (jnp.concatenate(ups, axis=1) if NI > 1 else ups[0])

    r2 = conv3x3(jnp.maximum(m, 0.0).astype(jnp.bfloat16), wrs_ref[2], 3)
    r2 = conv3x3(jnp.maximum(r2, 0.0).astype(jnp.bfloat16), wrs_ref[3], 4)
    out = m + r2
    for i in range(NI):
        o_ref[i] = out[:, i * HW:(i + 1) * HW]


def kernel(f, pm, convFS_w, convFS_b,
           resFS_conv1_w, resFS_conv1_b, resFS_conv2_w, resFS_conv2_b,
           resMM_conv1_w, resMM_conv1_b, resMM_conv2_w, resMM_conv2_b):
    N, Cin, H, W = f.shape
    _, C, h, w = pm.shape
    HW, hw = H * W, h * w
    NI = 8 if N % 8 == 0 else (4 if N % 4 == 0 else (2 if N % 2 == 0 else 1))

    f_flat = f.reshape(N, Cin, HW)
    pm_flat = pm.reshape(N, C, hw)

    wfs = (jnp.transpose(convFS_w, (0, 3, 1, 2))
           .reshape(3 * C, 3 * Cin).astype(jnp.bfloat16))
    wrs = (jnp.transpose(
        jnp.stack([resFS_conv1_w, resFS_conv2_w, resMM_conv1_w, resMM_conv2_w]),
        (0, 1, 4, 2, 3)).reshape(4, 3 * C, 3 * C).astype(jnp.bfloat16))
    bcols = jnp.stack([convFS_b, resFS_conv1_b, resFS_conv2_b,
                       resMM_conv1_b, resMM_conv2_b], axis=1)

    aht = _interp_mat_np(H, h).T
    awt = _interp_mat_np(W, w).T
    kup = jnp.asarray(
        (aht[:, None, :, None] * awt[None, :, None, :]).reshape(hw, HW))

    whole = lambda shp: pl.BlockSpec(shp, lambda i: (0,) * len(shp))

    out = pl.pallas_call(
        functools.partial(_refine_kernel, H=H, W=W, NI=NI),
        out_shape=jax.ShapeDtypeStruct((N, C, HW), jnp.float32),
        grid=(N // NI,),
        in_specs=[
            pl.BlockSpec((NI, Cin, HW), lambda i: (i, 0, 0)),
            pl.BlockSpec((NI, C, hw), lambda i: (i, 0, 0)),
            whole((3 * C, 3 * Cin)),
            whole((4, 3 * C, 3 * C)),
            whole((C, 5)),
            whole((hw, HW)),
        ],
        out_specs=pl.BlockSpec((NI, C, HW), lambda i: (i, 0, 0)),
        compiler_params=pltpu.CompilerParams(
            dimension_semantics=("arbitrary",),
            vmem_limit_bytes=_VMEM_LIMIT),
    )(f_flat, pm_flat, wfs, wrs, bcols, kup)
    return out.reshape(N, C, H, W)
